# Initial kernel scaffold; baseline (speedup 1.0000x reference)
import jax
import jax.numpy as jnp
from jax.experimental import pallas as pl


def kernel(batch_box_preds, batch_cls_preds):
    raise NotImplementedError("write your pallas kernel here")


# speedup vs baseline: 74.1882x; 74.1882x over previous
import jax
import jax.numpy as jnp
from jax import lax
from jax.experimental import pallas as pl
from jax.experimental.pallas import tpu as pltpu

_N = 5000
_NP = 5120
_K = 4096
_T = 512
_NBR = _NP // _T
_NBK = _K // _T
_POST = 500
_TH = 0.7
_HI = lax.Precision.HIGHEST
_F32 = jnp.float32


def _iou_gt(x1c, x2c, y1c, y2c, ac, x1r, x2r, y1r, y2r, ar):
    ix = jnp.maximum(jnp.minimum(x2c, x2r) - jnp.maximum(x1c, x1r), 0.0)
    iy = jnp.maximum(jnp.minimum(y2c, y2r) - jnp.maximum(y1c, y1r), 0.0)
    inter = ix * iy
    iou = inter / (ac + ar - inter + 1e-6)
    return iou > _TH


def _row_coords(dsT, base):
    xr = dsT[0:1, pl.ds(base, _T)]
    yr = dsT[1:2, pl.ds(base, _T)]
    dxr = dsT[3:4, pl.ds(base, _T)]
    dyr = dsT[4:5, pl.ds(base, _T)]
    x1r = xr - dxr * 0.5
    x2r = xr + dxr * 0.5
    y1r = yr - dyr * 0.5
    y2r = yr + dyr * 0.5
    ar = (x2r - x1r) * (y2r - y1r)
    return x1r, x2r, y1r, y2r, ar


def _nms_kernel(box_ref, boxT_ref, cls_ref, clsT_ref, out_ref,
                draw, drawT, rank, ds, dsT, keep, cnt):
    out_ref[...] = jnp.zeros((1, _T, 16), _F32)
    keep[...] = jnp.zeros((_K, 1), _F32)
    cnt[0] = 0.0

    cls_ = cls_ref[0]
    clsT = clsT_ref[0]
    c0 = cls_[:, 0:1]
    c1 = cls_[:, 1:2]
    c2 = cls_[:, 2:3]
    s_col = jnp.maximum(jnp.maximum(c0, c1), c2)
    l_col = jnp.where(c1 > c0, 1.0, 0.0)
    l_col = jnp.where(c2 > jnp.maximum(c0, c1), 2.0, l_col) + 1.0
    r0 = clsT[0:1, :]
    r1 = clsT[1:2, :]
    r2 = clsT[2:3, :]
    s_row = jnp.maximum(jnp.maximum(r0, r1), r2)
    l_row = jnp.where(r1 > r0, 1.0, 0.0)
    l_row = jnp.where(r2 > jnp.maximum(r0, r1), 2.0, l_row) + 1.0

    draw[:, 0:7] = box_ref[0]
    draw[:, 7:8] = s_col
    draw[:, 8:9] = l_col
    draw[:, 9:16] = jnp.zeros((_NP, 7), _F32)
    drawT[0:7, :] = boxT_ref[0]
    drawT[7:8, :] = s_row
    drawT[8:9, :] = l_row
    drawT[9:16, :] = jnp.zeros((7, _NP), _F32)

    rank[...] = jnp.zeros((_NP, 1), _F32)
    dmat = (lax.broadcasted_iota(jnp.int32, (_T, _T), 1)
            - lax.broadcasted_iota(jnp.int32, (_T, _T), 0))

    def rank_bj(bj, _):
        srow_j = drawT[7:8, pl.ds(bj * _T, _T)]

        def rank_bi(bi, _):
            scol_i = draw[pl.ds(bi * _T, _T), 7:8]
            gt = srow_j > scol_i
            eq = srow_j == scol_i
            jlt = dmat < (bi - bj) * _T
            m = gt | (eq & jlt)
            contrib = jnp.sum(jnp.where(m, 1.0, 0.0), axis=1, keepdims=True)
            rank[pl.ds(bi * _T, _T), :] += contrib
            return 0

        lax.fori_loop(0, _NBR, rank_bi, 0, unroll=False)
        return 0

    lax.fori_loop(0, _NBR, rank_bj, 0, unroll=False)

    iot0 = lax.broadcasted_iota(jnp.int32, (_T, _T), 0)
    iot1 = lax.broadcasted_iota(jnp.int32, (_T, _T), 1)
    low_tri = jnp.where(iot0 > iot1, 1.0, 0.0)
    o_iota = iot1.astype(_F32)

    for b in range(_NBK):
        base = b * _T

        @pl.when(cnt[0] < float(_POST))
        def _process():
            ds[base:base + _T, :] = jnp.zeros((_T, 16), _F32)
            dsT[:, base:base + _T] = jnp.zeros((16, _T), _F32)

            def sortb(bi, _):
                rk = rank[pl.ds(bi * _T, _T), :]
                pf = jnp.where(rk == (o_iota + float(base)), 1.0, 0.0)
                dblk = draw[pl.ds(bi * _T, _T), :]
                ds[base:base + _T, :] += lax.dot_general(
                    pf, dblk, (((0,), (0,)), ((), ())), precision=_HI)
                dTblk = drawT[:, pl.ds(bi * _T, _T)]
                dsT[:, base:base + _T] += lax.dot_general(
                    dTblk, pf, (((1,), (0,)), ((), ())), precision=_HI)
                return 0

            lax.fori_loop(0, _NBR, sortb, 0, unroll=False)

            dbv = ds[base:base + _T, :]
            xc = dbv[:, 0:1]
            yc = dbv[:, 1:2]
            dxc = dbv[:, 3:4]
            dyc = dbv[:, 4:5]
            x1c = xc - dxc * 0.5
            x2c = xc + dxc * 0.5
            y1c = yc - dyc * 0.5
            y2c = yc + dyc * 0.5
            ac = (x2c - x1c) * (y2c - y1c)

            def crossa(a, sup):
                x1r, x2r, y1r, y2r, ar = _row_coords(dsT, a * _T)
                m = _iou_gt(x1c, x2c, y1c, y2c, ac, x1r, x2r, y1r, y2r, ar)
                ka = keep[pl.ds(a * _T, _T), :]
                return sup + lax.dot_general(
                    jnp.where(m, 1.0, 0.0), ka, (((1,), (0,)), ((), ())),
                    precision=_HI)

            sup = lax.fori_loop(0, b, crossa, jnp.zeros((_T, 1), _F32),
                                unroll=False)
            initf = jnp.where(sup == 0.0, 1.0, 0.0)

            x1r, x2r, y1r, y2r, ar = _row_coords(dsT, base)
            mlow = _iou_gt(x1c, x2c, y1c, y2c, ac, x1r, x2r, y1r, y2r, ar)
            mlow = mlow & (iot0 > iot1)
            mf = jnp.where(mlow, 1.0, 0.0)

            def jcond(st):
                return jnp.logical_not(st[1])

            def jbody(st):
                a, _ = st
                supin = lax.dot_general(mf, a, (((1,), (0,)), ((), ())),
                                        precision=_HI)
                a2 = initf * jnp.where(supin == 0.0, 1.0, 0.0)
                return (a2, jnp.all(a2 == a))

            af, _ = lax.while_loop(jcond, jbody, (initf, jnp.array(False)))
            keep[base:base + _T, :] = af

            c_before = cnt[0]
            pcol = lax.dot_general(low_tri, af, (((1,), (0,)), ((), ())),
                                   precision=_HI) + c_before
            qf = jnp.where((af > 0.0) & (pcol == o_iota), 1.0, 0.0)
            out_ref[0] += lax.dot_general(qf, dbv, (((0,), (0,)), ((), ())),
                                          precision=_HI)
            cnt[0] = c_before + jnp.sum(af)


def _run(box, boxT, cls_, clsT, interpret=False):
    b = box.shape[0]
    return pl.pallas_call(
        _nms_kernel,
        grid=(b,),
        in_specs=[
            pl.BlockSpec((1, _NP, 7), lambda i: (i, 0, 0)),
            pl.BlockSpec((1, 7, _NP), lambda i: (i, 0, 0)),
            pl.BlockSpec((1, _NP, 3), lambda i: (i, 0, 0)),
            pl.BlockSpec((1, 3, _NP), lambda i: (i, 0, 0)),
        ],
        out_specs=pl.BlockSpec((1, _T, 16), lambda i: (i, 0, 0)),
        out_shape=jax.ShapeDtypeStruct((b, _T, 16), _F32),
        scratch_shapes=[
            pltpu.VMEM((_NP, 16), _F32),
            pltpu.VMEM((16, _NP), _F32),
            pltpu.VMEM((_NP, 1), _F32),
            pltpu.VMEM((_K, 16), _F32),
            pltpu.VMEM((16, _K), _F32),
            pltpu.VMEM((_K, 1), _F32),
            pltpu.SMEM((1,), _F32),
        ],
        interpret=interpret,
    )(box, boxT, cls_, clsT)


def kernel(batch_box_preds, batch_cls_preds):
    pad = _NP - batch_box_preds.shape[1]
    box = jnp.pad(batch_box_preds, ((0, 0), (0, pad), (0, 0)))
    cls_ = jnp.pad(batch_cls_preds, ((0, 0), (0, pad), (0, 0)),
                   constant_values=-1e30)
    boxT = jnp.transpose(box, (0, 2, 1))
    clsT = jnp.transpose(cls_, (0, 2, 1))
    out = _run(box, boxT, cls_, clsT)
    rois = out[:, :_POST, 0:7]
    roi_scores = out[:, :_POST, 7]
    roi_labels = out[:, :_POST, 8].astype(jnp.int32)
    return rois, roi_scores, roi_labels


# speedup vs baseline: 95.4937x; 1.2872x over previous
import jax
import jax.numpy as jnp
from jax import lax
from jax.experimental import pallas as pl
from jax.experimental.pallas import tpu as pltpu

_N = 5000
_NP = 5120
_K = 4096
_T = 512
_NBR = _NP // _T
_NBK = _K // _T
_POST = 500
_TH = 0.7
_HI = lax.Precision.HIGHEST
_B3 = lax.Precision.HIGHEST
_DF = lax.Precision.DEFAULT
_F32 = jnp.float32


def _iou_gt(x1c, x2c, y1c, y2c, ac, x1r, x2r, y1r, y2r, ar):
    ix = jnp.maximum(jnp.minimum(x2c, x2r) - jnp.maximum(x1c, x1r), 0.0)
    iy = jnp.maximum(jnp.minimum(y2c, y2r) - jnp.maximum(y1c, y1r), 0.0)
    inter = ix * iy
    iou = inter / (ac + ar - inter + 1e-6)
    return iou > _TH


def _row_coords(dsT, base):
    xr = dsT[0:1, pl.ds(base, _T)]
    yr = dsT[1:2, pl.ds(base, _T)]
    dxr = dsT[3:4, pl.ds(base, _T)]
    dyr = dsT[4:5, pl.ds(base, _T)]
    x1r = xr - dxr * 0.5
    x2r = xr + dxr * 0.5
    y1r = yr - dyr * 0.5
    y2r = yr + dyr * 0.5
    ar = (x2r - x1r) * (y2r - y1r)
    return x1r, x2r, y1r, y2r, ar


def _nms_kernel(box_ref, boxT_ref, cls_ref, clsT_ref, out_ref,
                draw, drawT, rank, ds, dsT, keep, cnt):
    out_ref[...] = jnp.zeros((1, _T, 16), _F32)
    keep[...] = jnp.zeros((_K, 1), _F32)
    cnt[0] = 0.0

    cls_ = cls_ref[0]
    clsT = clsT_ref[0]
    c0 = cls_[:, 0:1]
    c1 = cls_[:, 1:2]
    c2 = cls_[:, 2:3]
    s_col = jnp.maximum(jnp.maximum(c0, c1), c2)
    l_col = jnp.where(c1 > c0, 1.0, 0.0)
    l_col = jnp.where(c2 > jnp.maximum(c0, c1), 2.0, l_col) + 1.0
    r0 = clsT[0:1, :]
    r1 = clsT[1:2, :]
    r2 = clsT[2:3, :]
    s_row = jnp.maximum(jnp.maximum(r0, r1), r2)
    l_row = jnp.where(r1 > r0, 1.0, 0.0)
    l_row = jnp.where(r2 > jnp.maximum(r0, r1), 2.0, l_row) + 1.0

    draw[:, 0:7] = box_ref[0]
    draw[:, 7:8] = s_col
    draw[:, 8:9] = l_col
    draw[:, 9:16] = jnp.zeros((_NP, 7), _F32)
    drawT[0:7, :] = boxT_ref[0]
    drawT[7:8, :] = s_row
    drawT[8:9, :] = l_row
    drawT[9:16, :] = jnp.zeros((7, _NP), _F32)

    iot0 = lax.broadcasted_iota(jnp.int32, (_T, _T), 0)
    iot1 = lax.broadcasted_iota(jnp.int32, (_T, _T), 1)
    low_tri = jnp.where(iot0 > iot1, 1.0, 0.0)
    o_iota = iot1.astype(_F32)

    def rank_bi(bi, _):
        scol_i = draw[pl.ds(bi * _T, _T), 7:8]

        def ge_bj(bj, acc):
            srow_j = drawT[7:8, pl.ds(bj * _T, _T)]
            return acc + jnp.where(srow_j >= scol_i, 1.0, 0.0)

        def gt_bj(bj, acc):
            srow_j = drawT[7:8, pl.ds(bj * _T, _T)]
            return acc + jnp.where(srow_j > scol_i, 1.0, 0.0)

        acc = lax.fori_loop(0, bi, ge_bj, jnp.zeros((_T, _T), _F32),
                            unroll=False)
        srow_d = drawT[7:8, pl.ds(bi * _T, _T)]
        m = (srow_d > scol_i) | ((srow_d == scol_i) & (iot1 < iot0))
        acc = acc + jnp.where(m, 1.0, 0.0)
        acc = lax.fori_loop(bi + 1, _NBR, gt_bj, acc, unroll=False)
        rank[pl.ds(bi * _T, _T), :] = jnp.sum(acc, axis=1, keepdims=True)
        return 0

    lax.fori_loop(0, _NBR, rank_bi, 0, unroll=False)

    for b in range(_NBK):
        base = b * _T

        @pl.when(cnt[0] < float(_POST))
        def _process():
            ds[base:base + _T, :] = jnp.zeros((_T, 16), _F32)
            dsT[:, base:base + _T] = jnp.zeros((16, _T), _F32)

            def sortb(bi, _):
                rk = rank[pl.ds(bi * _T, _T), :]
                pf = jnp.where(rk == (o_iota + float(base)), 1.0, 0.0)
                dblk = draw[pl.ds(bi * _T, _T), :]
                ds[base:base + _T, :] += lax.dot_general(
                    pf, dblk, (((0,), (0,)), ((), ())), precision=_B3)
                dTblk = drawT[:, pl.ds(bi * _T, _T)]
                dsT[:, base:base + _T] += lax.dot_general(
                    dTblk, pf, (((1,), (0,)), ((), ())), precision=_B3)
                return 0

            lax.fori_loop(0, _NBR, sortb, 0, unroll=False)

            dbv = ds[base:base + _T, :]
            xc = dbv[:, 0:1]
            yc = dbv[:, 1:2]
            dxc = dbv[:, 3:4]
            dyc = dbv[:, 4:5]
            x1c = xc - dxc * 0.5
            x2c = xc + dxc * 0.5
            y1c = yc - dyc * 0.5
            y2c = yc + dyc * 0.5
            ac = (x2c - x1c) * (y2c - y1c)

            def crossa(a, sup):
                x1r, x2r, y1r, y2r, ar = _row_coords(dsT, a * _T)
                m = _iou_gt(x1c, x2c, y1c, y2c, ac, x1r, x2r, y1r, y2r, ar)
                ka = keep[pl.ds(a * _T, _T), :]
                return sup + lax.dot_general(
                    jnp.where(m, 1.0, 0.0), ka, (((1,), (0,)), ((), ())),
                    precision=_DF)

            sup = lax.fori_loop(0, b, crossa, jnp.zeros((_T, 1), _F32),
                                unroll=False)
            initf = jnp.where(sup == 0.0, 1.0, 0.0)

            x1r, x2r, y1r, y2r, ar = _row_coords(dsT, base)
            mlow = _iou_gt(x1c, x2c, y1c, y2c, ac, x1r, x2r, y1r, y2r, ar)
            mlow = mlow & (iot0 > iot1)
            mf = jnp.where(mlow, 1.0, 0.0)

            def jcond(st):
                return jnp.logical_not(st[1])

            def jbody(st):
                a, _ = st
                supin = lax.dot_general(mf, a, (((1,), (0,)), ((), ())),
                                        precision=_DF)
                a2 = initf * jnp.where(supin == 0.0, 1.0, 0.0)
                return (a2, jnp.all(a2 == a))

            af, _ = lax.while_loop(jcond, jbody, (initf, jnp.array(False)))
            keep[base:base + _T, :] = af

            c_before = cnt[0]
            pcol = lax.dot_general(low_tri, af, (((1,), (0,)), ((), ())),
                                   precision=_DF) + c_before
            qf = jnp.where((af > 0.0) & (pcol == o_iota), 1.0, 0.0)
            out_ref[0] += lax.dot_general(qf, dbv, (((0,), (0,)), ((), ())),
                                          precision=_B3)
            cnt[0] = c_before + jnp.sum(af)


def _run(box, boxT, cls_, clsT, interpret=False):
    b = box.shape[0]
    return pl.pallas_call(
        _nms_kernel,
        grid=(b,),
        in_specs=[
            pl.BlockSpec((1, _NP, 7), lambda i: (i, 0, 0)),
            pl.BlockSpec((1, 7, _NP), lambda i: (i, 0, 0)),
            pl.BlockSpec((1, _NP, 3), lambda i: (i, 0, 0)),
            pl.BlockSpec((1, 3, _NP), lambda i: (i, 0, 0)),
        ],
        out_specs=pl.BlockSpec((1, _T, 16), lambda i: (i, 0, 0)),
        out_shape=jax.ShapeDtypeStruct((b, _T, 16), _F32),
        compiler_params=pltpu.CompilerParams(
            dimension_semantics=("parallel",)),
        scratch_shapes=[
            pltpu.VMEM((_NP, 16), _F32),
            pltpu.VMEM((16, _NP), _F32),
            pltpu.VMEM((_NP, 1), _F32),
            pltpu.VMEM((_K, 16), _F32),
            pltpu.VMEM((16, _K), _F32),
            pltpu.VMEM((_K, 1), _F32),
            pltpu.SMEM((1,), _F32),
        ],
        interpret=interpret,
    )(box, boxT, cls_, clsT)


def kernel(batch_box_preds, batch_cls_preds):
    pad = _NP - batch_box_preds.shape[1]
    box = jnp.pad(batch_box_preds, ((0, 0), (0, pad), (0, 0)))
    cls_ = jnp.pad(batch_cls_preds, ((0, 0), (0, pad), (0, 0)),
                   constant_values=-1e30)
    boxT = jnp.transpose(box, (0, 2, 1))
    clsT = jnp.transpose(cls_, (0, 2, 1))
    out = _run(box, boxT, cls_, clsT)
    rois = out[:, :_POST, 0:7]
    roi_scores = out[:, :_POST, 7]
    roi_labels = out[:, :_POST, 8].astype(jnp.int32)
    return rois, roi_scores, roi_labels


# speedup vs baseline: 182.3932x; 1.9100x over previous
import jax
import jax.numpy as jnp
from jax import lax
from jax.experimental import pallas as pl
from jax.experimental.pallas import tpu as pltpu

_N = 5000
_NP = 5120
_K = 4096
_T = 512
_NBR = _NP // _T
_NBK = _K // _T
_POST = 500
_TH = 0.7
_HI = lax.Precision.HIGHEST
_B3 = lax.Precision.HIGHEST
_DF = lax.Precision.DEFAULT
_F32 = jnp.float32


def _iou_gt(x1c, x2c, y1c, y2c, ac, x1r, x2r, y1r, y2r, ar):
    ix = jnp.maximum(jnp.minimum(x2c, x2r) - jnp.maximum(x1c, x1r), 0.0)
    iy = jnp.maximum(jnp.minimum(y2c, y2r) - jnp.maximum(y1c, y1r), 0.0)
    inter = ix * iy
    iou = inter / (ac + ar - inter + 1e-6)
    return iou > _TH


def _row_coords(dsT, base):
    xr = dsT[0:1, pl.ds(base, _T)]
    yr = dsT[1:2, pl.ds(base, _T)]
    dxr = dsT[3:4, pl.ds(base, _T)]
    dyr = dsT[4:5, pl.ds(base, _T)]
    x1r = xr - dxr * 0.5
    x2r = xr + dxr * 0.5
    y1r = yr - dyr * 0.5
    y2r = yr + dyr * 0.5
    ar = (x2r - x1r) * (y2r - y1r)
    return x1r, x2r, y1r, y2r, ar


def _nms_kernel(boxT_ref, cls_ref, clsT_ref, out_ref,
                scol, drawT, rank, dsT, keep, cnt):
    out_ref[...] = jnp.zeros((1, 16, _T), _F32)
    keep[...] = jnp.zeros((_K, 1), _F32)
    cnt[0] = 0.0

    cls_ = cls_ref[0]
    clsT = clsT_ref[0]
    c0 = cls_[:, 0:1]
    c1 = cls_[:, 1:2]
    c2 = cls_[:, 2:3]
    s_col = jnp.maximum(jnp.maximum(c0, c1), c2)
    r0 = clsT[0:1, :]
    r1 = clsT[1:2, :]
    r2 = clsT[2:3, :]
    s_row = jnp.maximum(jnp.maximum(r0, r1), r2)
    l_row = jnp.where(r1 > r0, 1.0, 0.0)
    l_row = jnp.where(r2 > jnp.maximum(r0, r1), 2.0, l_row) + 1.0

    scol[...] = s_col
    drawT[0:7, :] = boxT_ref[0]
    drawT[7:8, :] = s_row
    drawT[8:9, :] = l_row
    drawT[9:16, :] = jnp.zeros((7, _NP), _F32)

    iot0 = lax.broadcasted_iota(jnp.int32, (_T, _T), 0)
    iot1 = lax.broadcasted_iota(jnp.int32, (_T, _T), 1)
    low_tri = jnp.where(iot0 > iot1, 1.0, 0.0)
    o_iota = iot1.astype(_F32)
    o_iota0 = iot0.astype(_F32)

    def rank_bi(bi, _):
        scol_i = scol[pl.ds(bi * _T, _T), :]

        def ge_bj(bj, acc):
            srow_j = drawT[7:8, pl.ds(bj * _T, _T)]
            return acc + jnp.where(srow_j >= scol_i, 1.0, 0.0)

        def gt_bj(bj, acc):
            srow_j = drawT[7:8, pl.ds(bj * _T, _T)]
            return acc + jnp.where(srow_j > scol_i, 1.0, 0.0)

        acc = lax.fori_loop(0, bi, ge_bj, jnp.zeros((_T, _T), _F32),
                            unroll=False)
        srow_d = drawT[7:8, pl.ds(bi * _T, _T)]
        m = (srow_d > scol_i) | ((srow_d == scol_i) & (iot1 < iot0))
        acc = acc + jnp.where(m, 1.0, 0.0)
        acc = lax.fori_loop(bi + 1, _NBR, gt_bj, acc, unroll=False)
        rank[pl.ds(bi * _T, _T), :] = jnp.sum(acc, axis=1, keepdims=True)
        return 0

    lax.fori_loop(0, _NBR, rank_bi, 0, unroll=False)

    for b in range(_NBK):
        base = b * _T

        @pl.when(cnt[0] < float(_POST))
        def _process():
            dsT[:, base:base + _T] = jnp.zeros((16, _T), _F32)

            def sortb(bi, _):
                rk = rank[pl.ds(bi * _T, _T), :]
                pf = jnp.where(rk == (o_iota + float(base)), 1.0, 0.0)
                dTblk = drawT[:, pl.ds(bi * _T, _T)]
                dsT[:, base:base + _T] += lax.dot_general(
                    dTblk, pf, (((1,), (0,)), ((), ())), precision=_B3)
                return 0

            lax.fori_loop(0, _NBR, sortb, 0, unroll=False)

            dcols = jnp.transpose(dsT[0:8, base:base + _T])
            xc = dcols[:, 0:1]
            yc = dcols[:, 1:2]
            dxc = dcols[:, 3:4]
            dyc = dcols[:, 4:5]
            x1c = xc - dxc * 0.5
            x2c = xc + dxc * 0.5
            y1c = yc - dyc * 0.5
            y2c = yc + dyc * 0.5
            ac = (x2c - x1c) * (y2c - y1c)

            def crossa(a, sup):
                x1r, x2r, y1r, y2r, ar = _row_coords(dsT, a * _T)
                m = _iou_gt(x1c, x2c, y1c, y2c, ac, x1r, x2r, y1r, y2r, ar)
                ka = keep[pl.ds(a * _T, _T), :]
                return sup + lax.dot_general(
                    jnp.where(m, 1.0, 0.0), ka, (((1,), (0,)), ((), ())),
                    precision=_DF)

            sup = lax.fori_loop(0, b, crossa, jnp.zeros((_T, 1), _F32),
                                unroll=False)
            initf = jnp.where(sup == 0.0, 1.0, 0.0)

            x1r, x2r, y1r, y2r, ar = _row_coords(dsT, base)
            mlow = _iou_gt(x1c, x2c, y1c, y2c, ac, x1r, x2r, y1r, y2r, ar)
            mlow = mlow & (iot0 > iot1)
            mf = jnp.where(mlow, 1.0, 0.0)

            def jcond(st):
                return jnp.logical_not(st[1])

            def jbody(st):
                a, _ = st
                supin = lax.dot_general(mf, a, (((1,), (0,)), ((), ())),
                                        precision=_DF)
                a2 = initf * jnp.where(supin == 0.0, 1.0, 0.0)
                return (a2, jnp.all(a2 == a))

            af, _ = lax.while_loop(jcond, jbody, (initf, jnp.array(False)))
            keep[base:base + _T, :] = af

            c_before = cnt[0]
            pcol = lax.dot_general(low_tri, af, (((1,), (0,)), ((), ())),
                                   precision=_DF) + c_before
            qf = jnp.where((af > 0.0) & (pcol == o_iota), 1.0, 0.0)
            out_ref[0] += lax.dot_general(
                dsT[:, base:base + _T], qf, (((1,), (0,)), ((), ())),
                precision=_B3)
            cnt[0] = c_before + jnp.sum(af)


def _run(boxT, cls_, clsT, interpret=False):
    b = boxT.shape[0]
    return pl.pallas_call(
        _nms_kernel,
        grid=(b,),
        in_specs=[
            pl.BlockSpec((1, 7, _NP), lambda i: (i, 0, 0)),
            pl.BlockSpec((1, _NP, 3), lambda i: (i, 0, 0)),
            pl.BlockSpec((1, 3, _NP), lambda i: (i, 0, 0)),
        ],
        out_specs=pl.BlockSpec((1, 16, _T), lambda i: (i, 0, 0)),
        out_shape=jax.ShapeDtypeStruct((b, 16, _T), _F32),
        compiler_params=pltpu.CompilerParams(
            dimension_semantics=("parallel",)),
        scratch_shapes=[
            pltpu.VMEM((_NP, 1), _F32),
            pltpu.VMEM((16, _NP), _F32),
            pltpu.VMEM((_NP, 1), _F32),
            pltpu.VMEM((16, _K), _F32),
            pltpu.VMEM((_K, 1), _F32),
            pltpu.SMEM((1,), _F32),
        ],
        interpret=interpret,
    )(boxT, cls_, clsT)


def kernel(batch_box_preds, batch_cls_preds):
    pad = _NP - batch_box_preds.shape[1]
    box = jnp.pad(batch_box_preds, ((0, 0), (0, pad), (0, 0)))
    cls_ = jnp.pad(batch_cls_preds, ((0, 0), (0, pad), (0, 0)),
                   constant_values=-1e30)
    boxT = jnp.transpose(box, (0, 2, 1))
    clsT = jnp.transpose(cls_, (0, 2, 1))
    outT = _run(boxT, cls_, clsT)
    out = jnp.transpose(outT, (0, 2, 1))
    rois = out[:, :_POST, 0:7]
    roi_scores = out[:, :_POST, 7]
    roi_labels = out[:, :_POST, 8].astype(jnp.int32)
    return rois, roi_scores, roi_labels


# speedup vs baseline: 216.8885x; 1.1891x over previous
import jax
import jax.numpy as jnp
from jax import lax
from jax.experimental import pallas as pl
from jax.experimental.pallas import tpu as pltpu

_N = 5000
_NP = 5120
_K = 4096
_T = 512
_NBR = _NP // _T
_NBK = _K // _T
_POST = 500
_TH = 0.7
_HI = lax.Precision.HIGHEST
_B3 = lax.Precision.HIGHEST
_DF = lax.Precision.DEFAULT
_F32 = jnp.float32


def _iou_gt(x1c, x2c, y1c, y2c, ac, x1r, x2r, y1r, y2r, ar):
    ix = jnp.maximum(jnp.minimum(x2c, x2r) - jnp.maximum(x1c, x1r), 0.0)
    iy = jnp.maximum(jnp.minimum(y2c, y2r) - jnp.maximum(y1c, y1r), 0.0)
    inter = ix * iy
    iou = inter / (ac + ar - inter + 1e-6)
    return iou > _TH


def _row_coords(dsT, base):
    xr = dsT[0:1, pl.ds(base, _T)]
    yr = dsT[1:2, pl.ds(base, _T)]
    dxr = dsT[3:4, pl.ds(base, _T)]
    dyr = dsT[4:5, pl.ds(base, _T)]
    x1r = xr - dxr * 0.5
    x2r = xr + dxr * 0.5
    y1r = yr - dyr * 0.5
    y2r = yr + dyr * 0.5
    ar = (x2r - x1r) * (y2r - y1r)
    return x1r, x2r, y1r, y2r, ar


def _nms_kernel(boxT_ref, cls_ref, clsT_ref, out_ref,
                scol, drawT, rank, dsT, keep, cmpm, cnt):
    out_ref[...] = jnp.zeros((1, 16, _T), _F32)
    keep[...] = jnp.zeros((_K, 1), _F32)
    cnt[0] = 0.0

    cls_ = cls_ref[0]
    clsT = clsT_ref[0]
    c0 = cls_[:, 0:1]
    c1 = cls_[:, 1:2]
    c2 = cls_[:, 2:3]
    s_col = jnp.maximum(jnp.maximum(c0, c1), c2)
    r0 = clsT[0:1, :]
    r1 = clsT[1:2, :]
    r2 = clsT[2:3, :]
    s_row = jnp.maximum(jnp.maximum(r0, r1), r2)
    l_row = jnp.where(r1 > r0, 1.0, 0.0)
    l_row = jnp.where(r2 > jnp.maximum(r0, r1), 2.0, l_row) + 1.0

    scol[...] = s_col
    drawT[0:7, :] = boxT_ref[0]
    drawT[7:8, :] = s_row
    drawT[8:9, :] = l_row
    drawT[9:16, :] = jnp.zeros((7, _NP), _F32)

    iot0 = lax.broadcasted_iota(jnp.int32, (_T, _T), 0)
    iot1 = lax.broadcasted_iota(jnp.int32, (_T, _T), 1)
    low_tri = jnp.where(iot0 > iot1, 1.0, 0.0)
    o_iota = iot1.astype(_F32)
    o_iota0 = iot0.astype(_F32)

    ones_np1 = jnp.ones((_NP, 1), _F32)

    def rank_bi(bi, _):
        scol_i = scol[pl.ds(bi * _T, _T), :]

        def ge_bj(bj, _):
            srow_j = drawT[7:8, pl.ds(bj * _T, _T)]
            cmpm[:, pl.ds(bj * _T, _T)] = jnp.where(srow_j >= scol_i, 1.0, 0.0)
            return 0

        def gt_bj(bj, _):
            srow_j = drawT[7:8, pl.ds(bj * _T, _T)]
            cmpm[:, pl.ds(bj * _T, _T)] = jnp.where(srow_j > scol_i, 1.0, 0.0)
            return 0

        lax.fori_loop(0, bi, ge_bj, 0, unroll=False)
        srow_d = drawT[7:8, pl.ds(bi * _T, _T)]
        m = (srow_d > scol_i) | ((srow_d == scol_i) & (iot1 < iot0))
        cmpm[:, pl.ds(bi * _T, _T)] = jnp.where(m, 1.0, 0.0)
        lax.fori_loop(bi + 1, _NBR, gt_bj, 0, unroll=False)
        rank[pl.ds(bi * _T, _T), :] = lax.dot_general(
            cmpm[...], ones_np1, (((1,), (0,)), ((), ())), precision=_DF)
        return 0

    lax.fori_loop(0, _NBR, rank_bi, 0, unroll=False)

    for b in range(_NBK):
        base = b * _T

        @pl.when(cnt[0] < float(_POST))
        def _process():
            dsT[:, base:base + _T] = jnp.zeros((16, _T), _F32)

            def sortb(bi, _):
                rk = rank[pl.ds(bi * _T, _T), :]
                pf = jnp.where(rk == (o_iota + float(base)), 1.0, 0.0)
                dTblk = drawT[:, pl.ds(bi * _T, _T)]
                dsT[:, base:base + _T] += lax.dot_general(
                    dTblk, pf, (((1,), (0,)), ((), ())), precision=_B3)
                return 0

            lax.fori_loop(0, _NBR, sortb, 0, unroll=False)

            dcols = jnp.transpose(dsT[0:8, base:base + _T])
            xc = dcols[:, 0:1]
            yc = dcols[:, 1:2]
            dxc = dcols[:, 3:4]
            dyc = dcols[:, 4:5]
            x1c = xc - dxc * 0.5
            x2c = xc + dxc * 0.5
            y1c = yc - dyc * 0.5
            y2c = yc + dyc * 0.5
            ac = (x2c - x1c) * (y2c - y1c)

            def crossa(a, sup):
                x1r, x2r, y1r, y2r, ar = _row_coords(dsT, a * _T)
                m = _iou_gt(x1c, x2c, y1c, y2c, ac, x1r, x2r, y1r, y2r, ar)
                ka = keep[pl.ds(a * _T, _T), :]
                return sup + lax.dot_general(
                    jnp.where(m, 1.0, 0.0), ka, (((1,), (0,)), ((), ())),
                    precision=_DF)

            sup = lax.fori_loop(0, b, crossa, jnp.zeros((_T, 1), _F32),
                                unroll=False)
            initf = jnp.where(sup == 0.0, 1.0, 0.0)

            x1r, x2r, y1r, y2r, ar = _row_coords(dsT, base)
            mlow = _iou_gt(x1c, x2c, y1c, y2c, ac, x1r, x2r, y1r, y2r, ar)
            mlow = mlow & (iot0 > iot1)
            mf = jnp.where(mlow, 1.0, 0.0)

            def jcond(st):
                return jnp.logical_not(st[1])

            def jbody(st):
                a, _ = st
                supin = lax.dot_general(mf, a, (((1,), (0,)), ((), ())),
                                        precision=_DF)
                a2 = initf * jnp.where(supin == 0.0, 1.0, 0.0)
                return (a2, jnp.all(a2 == a))

            af, _ = lax.while_loop(jcond, jbody, (initf, jnp.array(False)))
            keep[base:base + _T, :] = af

            c_before = cnt[0]
            pcol = lax.dot_general(low_tri, af, (((1,), (0,)), ((), ())),
                                   precision=_DF) + c_before
            qf = jnp.where((af > 0.0) & (pcol == o_iota), 1.0, 0.0)
            out_ref[0] += lax.dot_general(
                dsT[:, base:base + _T], qf, (((1,), (0,)), ((), ())),
                precision=_B3)
            cnt[0] = c_before + jnp.sum(af)


def _run(boxT, cls_, clsT, interpret=False):
    b = boxT.shape[0]
    return pl.pallas_call(
        _nms_kernel,
        grid=(b,),
        in_specs=[
            pl.BlockSpec((1, 7, _NP), lambda i: (i, 0, 0)),
            pl.BlockSpec((1, _NP, 3), lambda i: (i, 0, 0)),
            pl.BlockSpec((1, 3, _NP), lambda i: (i, 0, 0)),
        ],
        out_specs=pl.BlockSpec((1, 16, _T), lambda i: (i, 0, 0)),
        out_shape=jax.ShapeDtypeStruct((b, 16, _T), _F32),
        compiler_params=pltpu.CompilerParams(
            dimension_semantics=("parallel",)),
        scratch_shapes=[
            pltpu.VMEM((_NP, 1), _F32),
            pltpu.VMEM((16, _NP), _F32),
            pltpu.VMEM((_NP, 1), _F32),
            pltpu.VMEM((16, _K), _F32),
            pltpu.VMEM((_K, 1), _F32),
            pltpu.VMEM((_T, _NP), _F32),
            pltpu.SMEM((1,), _F32),
        ],
        interpret=interpret,
    )(boxT, cls_, clsT)


def kernel(batch_box_preds, batch_cls_preds):
    pad = _NP - batch_box_preds.shape[1]
    box = jnp.pad(batch_box_preds, ((0, 0), (0, pad), (0, 0)))
    cls_ = jnp.pad(batch_cls_preds, ((0, 0), (0, pad), (0, 0)),
                   constant_values=-1e30)
    boxT = jnp.transpose(box, (0, 2, 1))
    clsT = jnp.transpose(cls_, (0, 2, 1))
    outT = _run(boxT, cls_, clsT)
    out = jnp.transpose(outT, (0, 2, 1))
    rois = out[:, :_POST, 0:7]
    roi_scores = out[:, :_POST, 7]
    roi_labels = out[:, :_POST, 8].astype(jnp.int32)
    return rois, roi_scores, roi_labels
